# baseline (speedup 1.0000x reference)
import functools

import jax
import jax.numpy as jnp
from jax import lax
from jax.experimental import pallas as pl
from jax.experimental.pallas import tpu as pltpu
from jax.experimental.pallas import tpu_sc as plsc

N = 10000
E = 320000
D = 128
H = 128
C = 10
K = 3
G = 64

NC = 2
NS = 16
NW = NC * NS

CH = 80
TPC = 256
Q0 = 128
NBUF = 4
E_PAD = CH * TPC * NS
N_PAD = 10240
NPT = N_PAD // NS

_f32 = jnp.float32


def _make_propagate(with_deg: bool):
    mesh = plsc.VectorSubcoreMesh(core_axis_name="c", subcore_axis_name="s")

    out_type = [jax.ShapeDtypeStruct((NC, N_PAD, D), _f32)]
    scratch = (
        [pltpu.VMEM((2, CH), jnp.int32) for _ in range(NBUF)]
        + [pltpu.VMEM((CH, D), _f32) for _ in range(NBUF)]
        + [pltpu.VMEM_SHARED((N_PAD, D), _f32)]
        + [pltpu.SemaphoreType.DMA for _ in range(2 * NBUF)]
    )
    if with_deg:
        out_type.append(jax.ShapeDtypeStruct((NC, N_PAD), _f32))
        scratch += [
            pltpu.VMEM((CH,), _f32),
            pltpu.VMEM_SHARED((N_PAD,), _f32),
        ]

    def body(h_hbm, zeros2d, zeros1d, eidx_hbm, part_h, *rest):
        if with_deg:
            part_deg = rest[0]
            rest = rest[1:]
        ib = rest[0:NBUF]
        gb = rest[NBUF:2 * NBUF]
        acc = rest[2 * NBUF]
        smi = rest[2 * NBUF + 1:3 * NBUF + 1]
        smg = rest[3 * NBUF + 1:4 * NBUF + 1]
        if with_deg:
            ones_v = rest[4 * NBUF + 1]
            dacc = rest[4 * NBUF + 2]

        c = lax.axis_index("c")
        s = lax.axis_index("s")
        r0 = s * NPT
        start = lax.select(c == 0, 0, Q0)
        cnt = lax.select(c == 0, Q0, TPC - Q0)
        end = start + cnt

        pltpu.sync_copy(zeros2d.at[pl.ds(r0, NPT)], acc.at[pl.ds(r0, NPT)])
        if with_deg:
            pltpu.sync_copy(zeros1d.at[pl.ds(r0, NPT)], dacc.at[pl.ds(r0, NPT)])
            for k in range(CH // 16):
                ones_v[pl.ds(k * 16, 16)] = jnp.ones((16,), _f32)

        plsc.subcore_barrier()

        for p in range(NBUF - 1):
            @pl.when(cnt > p)
            def _(p=p):
                pltpu.sync_copy(eidx_hbm.at[s, start + p], ib[p])
                pltpu.async_copy(h_hbm.at[ib[p].at[0]], gb[p], smg[p])

        @pl.when(cnt > NBUF - 1)
        def _():
            pltpu.async_copy(eidx_hbm.at[s, start + NBUF - 1], ib[NBUF - 1],
                             smi[NBUF - 1])

        @pl.loop(0, cnt // NBUF)
        def _(i):
            for b in range(NBUF):
                j = start + NBUF * i + b
                nb = (b + NBUF - 1) % NBUF

                @pl.when(j + NBUF - 1 < end)
                def _():
                    pltpu.make_async_copy(eidx_hbm.at[s, j + NBUF - 1],
                                          ib[nb], smi[nb]).wait()
                    pltpu.async_copy(h_hbm.at[ib[nb].at[0]], gb[nb], smg[nb])

                pltpu.make_async_copy(h_hbm.at[ib[b].at[0]], gb[b],
                                      smg[b]).wait()
                pltpu.sync_copy(gb[b], acc.at[ib[b].at[1]], add=True)
                if with_deg:
                    pltpu.sync_copy(ones_v, dacc.at[ib[b].at[1]], add=True)

                @pl.when(j + NBUF < end)
                def _():
                    pltpu.async_copy(eidx_hbm.at[s, j + NBUF], ib[b], smi[b])

        plsc.subcore_barrier()

        pltpu.sync_copy(acc.at[pl.ds(r0, NPT)], part_h.at[c, pl.ds(r0, NPT)])
        if with_deg:
            pltpu.sync_copy(dacc.at[pl.ds(r0, NPT)], part_deg.at[c, pl.ds(r0, NPT)])

    return pl.kernel(body, out_type=out_type, mesh=mesh, scratch_types=scratch)


_prop_deg = _make_propagate(with_deg=True)
_prop = _make_propagate(with_deg=False)


def _combine1_body(ph, pd, hp, h_out, dinv_out):
    deg = pd[0, :, :] + pd[1, :, :] + 1.0
    dinv = 1.0 / deg
    dinv_out[...] = dinv
    h_out[...] = (ph[0, :, :] + ph[1, :, :] + hp[...]) * dinv


_combine1 = pl.pallas_call(
    _combine1_body,
    out_shape=[jax.ShapeDtypeStruct((N_PAD, D), _f32),
               jax.ShapeDtypeStruct((N_PAD, 1), _f32)],
)


def _combine_body(ph, dinv, hp, h_out):
    h_out[...] = (ph[0, :, :] + ph[1, :, :] + hp[...]) * dinv[...]


_combine = pl.pallas_call(
    _combine_body,
    out_shape=jax.ShapeDtypeStruct((N_PAD, D), _f32),
)


def _dot_t(a, b):
    return lax.dot_general(a, b, (((1,), (1,)), ((), ())),
                           preferred_element_type=_f32)


def _head_body(ph, dinv, hp, batch2, w1, b1, w2, b2, wc, bc, out):
    h = (ph[0, :, :] + ph[1, :, :] + hp[...]) * dinv[...]
    hid = jnp.maximum(_dot_t(h, w1[...]) + b1[...], 0.0)
    hid = _dot_t(hid, w2[...]) + b2[...]
    oh = (batch2[...] == lax.broadcasted_iota(jnp.int32, (1, G), 1)).astype(_f32)
    sums = lax.dot_general(oh, hid, (((0,), (0,)), ((), ())),
                           preferred_element_type=_f32)
    cnt = lax.dot_general(oh, jnp.ones((N_PAD, 1), _f32),
                          (((0,), (0,)), ((), ())),
                          preferred_element_type=_f32)
    pooled = sums * (1.0 / jnp.maximum(cnt, 1.0))
    out[...] = _dot_t(pooled, wc[...]) + bc[...]


_head = pl.pallas_call(
    _head_body,
    out_shape=jax.ShapeDtypeStruct((G, C), _f32),
)


def kernel(x, edge_index, batch, fc1_w, fc1_b, fc2_w, fc2_b, cls_w, cls_b):
    i32 = jnp.int32
    row = edge_index[0]
    col = edge_index[1]
    pad_iota = jnp.arange(E_PAD - E, dtype=i32)
    row_p = jnp.concatenate([row, N + pad_iota % (N_PAD - N)])
    col_p = jnp.concatenate([col, pad_iota % N])
    row_p = row_p.reshape(NS, TPC, CH)
    col_p = col_p.reshape(NS, TPC, CH)
    eidx = jnp.stack([col_p, row_p], axis=2)

    x_pad = jnp.pad(x, ((0, N_PAD - N), (0, 0)))
    zeros2d = jnp.zeros((N_PAD, D), _f32)
    zeros1d = jnp.zeros((N_PAD,), _f32)
    batch2 = jnp.concatenate([batch, jnp.full((N_PAD - N,), G, i32)])
    batch2 = batch2.reshape(N_PAD, 1)

    part_h, part_deg = _prop_deg(x_pad, zeros2d, zeros1d, eidx)
    h, dinv = _combine1(part_h, part_deg.reshape(NC, N_PAD, 1), x_pad)
    for _ in range(K - 2):
        (part_h,) = _prop(h, zeros2d, zeros1d, eidx)
        h = _combine(part_h, dinv, h)
    (part_h,) = _prop(h, zeros2d, zeros1d, eidx)
    return _head(part_h, dinv, h, batch2,
                 fc1_w, fc1_b, fc2_w, fc2_b, cls_w, cls_b)


# speedup vs baseline: 1.1387x; 1.1387x over previous
import functools

import jax
import jax.numpy as jnp
from jax import lax
from jax.experimental import pallas as pl
from jax.experimental.pallas import tpu as pltpu
from jax.experimental.pallas import tpu_sc as plsc

N = 10000
E = 320000
D = 128
H = 128
C = 10
K = 3
G = 64

NC = 2
NS = 16
NW = NC * NS

CH = 128
TPC = 160
Q0 = 80
E_PAD = CH * TPC * NS
N_PAD = 10240
NPT = N_PAD // NS

_f32 = jnp.float32


def _make_propagate(with_deg: bool):
    mesh = plsc.VectorSubcoreMesh(core_axis_name="c", subcore_axis_name="s")

    out_type = [jax.ShapeDtypeStruct((NC, N_PAD, D), _f32)]
    scratch = [
        pltpu.VMEM((2, CH), jnp.int32),
        pltpu.VMEM((2, CH), jnp.int32),
        pltpu.VMEM((CH, D), _f32),
        pltpu.VMEM((CH, D), _f32),
        pltpu.VMEM_SHARED((N_PAD, D), _f32),
        pltpu.SemaphoreType.DMA,
        pltpu.SemaphoreType.DMA,
        pltpu.SemaphoreType.DMA,
        pltpu.SemaphoreType.DMA,
    ]
    if with_deg:
        out_type.append(jax.ShapeDtypeStruct((NC, N_PAD), _f32))
        scratch += [
            pltpu.VMEM((CH,), _f32),
            pltpu.VMEM_SHARED((N_PAD,), _f32),
        ]

    def body(h_hbm, zeros2d, zeros1d, eidx_hbm, part_h, *rest):
        if with_deg:
            (part_deg, ib0, ib1, gb0, gb1, acc,
             smi0, smi1, smg0, smg1, ones_v, dacc) = rest
        else:
            ib0, ib1, gb0, gb1, acc, smi0, smi1, smg0, smg1 = rest
        ib = (ib0, ib1)
        gb = (gb0, gb1)
        smi = (smi0, smi1)
        smg = (smg0, smg1)

        c = lax.axis_index("c")
        s = lax.axis_index("s")
        r0 = s * NPT
        start = lax.select(c == 0, 0, Q0)
        cnt = lax.select(c == 0, Q0, TPC - Q0)
        end = start + cnt

        pltpu.sync_copy(zeros2d.at[pl.ds(r0, NPT)], acc.at[pl.ds(r0, NPT)])
        if with_deg:
            pltpu.sync_copy(zeros1d.at[pl.ds(r0, NPT)], dacc.at[pl.ds(r0, NPT)])
            for k in range(CH // 16):
                ones_v[pl.ds(k * 16, 16)] = jnp.ones((16,), _f32)

        plsc.subcore_barrier()

        @pl.when(cnt > 0)
        def _():
            pltpu.sync_copy(eidx_hbm.at[s, start], ib0)

            @pl.when(cnt > 1)
            def _():
                pltpu.async_copy(eidx_hbm.at[s, start + 1], ib1, smi1)

            pltpu.async_copy(h_hbm.at[ib0.at[0]], gb0, smg0)

        @pl.loop(0, cnt // 2)
        def _(i):
            for b in (0, 1):
                j = start + 2 * i + b

                @pl.when(j + 1 < end)
                def _():
                    pltpu.make_async_copy(eidx_hbm.at[s, j + 1],
                                          ib[1 - b], smi[1 - b]).wait()
                    pltpu.async_copy(h_hbm.at[ib[1 - b].at[0]], gb[1 - b],
                                     smg[1 - b])

                pltpu.make_async_copy(h_hbm.at[ib[b].at[0]], gb[b],
                                      smg[b]).wait()
                pltpu.sync_copy(gb[b], acc.at[ib[b].at[1]], add=True)
                if with_deg:
                    pltpu.sync_copy(ones_v, dacc.at[ib[b].at[1]], add=True)

                @pl.when(j + 2 < end)
                def _():
                    pltpu.async_copy(eidx_hbm.at[s, j + 2], ib[b], smi[b])

        plsc.subcore_barrier()

        pltpu.sync_copy(acc.at[pl.ds(r0, NPT)], part_h.at[c, pl.ds(r0, NPT)])
        if with_deg:
            pltpu.sync_copy(dacc.at[pl.ds(r0, NPT)], part_deg.at[c, pl.ds(r0, NPT)])

    return pl.kernel(body, out_type=out_type, mesh=mesh, scratch_types=scratch)


_prop_deg = _make_propagate(with_deg=True)
_prop = _make_propagate(with_deg=False)


def _combine1_body(ph, pd, hp, h_out, dinv_out):
    deg = pd[0, :, :] + pd[1, :, :] + 1.0
    dinv = 1.0 / deg
    dinv_out[...] = dinv
    h_out[...] = (ph[0, :, :] + ph[1, :, :] + hp[...]) * dinv


_combine1 = pl.pallas_call(
    _combine1_body,
    out_shape=[jax.ShapeDtypeStruct((N_PAD, D), _f32),
               jax.ShapeDtypeStruct((N_PAD, 1), _f32)],
)


def _combine_body(ph, dinv, hp, h_out):
    h_out[...] = (ph[0, :, :] + ph[1, :, :] + hp[...]) * dinv[...]


_combine = pl.pallas_call(
    _combine_body,
    out_shape=jax.ShapeDtypeStruct((N_PAD, D), _f32),
)


def _dot_t(a, b):
    return lax.dot_general(a, b, (((1,), (1,)), ((), ())),
                           preferred_element_type=_f32)


def _head_body(ph, dinv, hp, batch2, w1, b1, w2, b2, wc, bc, out):
    h = (ph[0, :, :] + ph[1, :, :] + hp[...]) * dinv[...]
    hid = jnp.maximum(_dot_t(h, w1[...]) + b1[...], 0.0)
    hid = _dot_t(hid, w2[...]) + b2[...]
    oh = (batch2[...] == lax.broadcasted_iota(jnp.int32, (1, G), 1)).astype(_f32)
    sums = lax.dot_general(oh, hid, (((0,), (0,)), ((), ())),
                           preferred_element_type=_f32)
    cnt = lax.dot_general(oh, jnp.ones((N_PAD, 1), _f32),
                          (((0,), (0,)), ((), ())),
                          preferred_element_type=_f32)
    pooled = sums * (1.0 / jnp.maximum(cnt, 1.0))
    out[...] = _dot_t(pooled, wc[...]) + bc[...]


_head = pl.pallas_call(
    _head_body,
    out_shape=jax.ShapeDtypeStruct((G, C), _f32),
)


def kernel(x, edge_index, batch, fc1_w, fc1_b, fc2_w, fc2_b, cls_w, cls_b):
    i32 = jnp.int32
    row = edge_index[0]
    col = edge_index[1]
    pad_iota = jnp.arange(E_PAD - E, dtype=i32)
    row_p = jnp.concatenate([row, N + pad_iota % (N_PAD - N)])
    col_p = jnp.concatenate([col, pad_iota % N])
    row_p = row_p.reshape(NS, TPC, CH)
    col_p = col_p.reshape(NS, TPC, CH)
    eidx = jnp.stack([col_p, row_p], axis=2)

    x_pad = jnp.pad(x, ((0, N_PAD - N), (0, 0)))
    zeros2d = jnp.zeros((N_PAD, D), _f32)
    zeros1d = jnp.zeros((N_PAD,), _f32)
    batch2 = jnp.concatenate([batch, jnp.full((N_PAD - N,), G, i32)])
    batch2 = batch2.reshape(N_PAD, 1)

    part_h, part_deg = _prop_deg(x_pad, zeros2d, zeros1d, eidx)
    h, dinv = _combine1(part_h, part_deg.reshape(NC, N_PAD, 1), x_pad)
    for _ in range(K - 2):
        (part_h,) = _prop(h, zeros2d, zeros1d, eidx)
        h = _combine(part_h, dinv, h)
    (part_h,) = _prop(h, zeros2d, zeros1d, eidx)
    return _head(part_h, dinv, h, batch2,
                 fc1_w, fc1_b, fc2_w, fc2_b, cls_w, cls_b)
